# Initial kernel scaffold; baseline (speedup 1.0000x reference)
import jax
import jax.numpy as jnp
from jax.experimental import pallas as pl


def kernel(apiid, interval, apiid_table, interval_table, ln_gamma, ln_beta):
    raise NotImplementedError("write your pallas kernel here")


# speedup vs baseline: 1.0157x; 1.0157x over previous
import functools

import jax
import jax.numpy as jnp
from jax import lax
from jax.experimental import pallas as pl
from jax.experimental.pallas import tpu as pltpu
from jax.experimental.pallas import tpu_sc as plsc

EPS = 1e-12
D = 64
L = 16
CHUNK = 128


def _sc_embed_ln(n_rows, n_small, num_workers):
    per_w = n_rows // num_workers
    n_chunks = per_w // CHUNK
    mesh = plsc.VectorSubcoreMesh(core_axis_name="c", subcore_axis_name="s")

    @functools.partial(
        pl.kernel,
        mesh=mesh,
        out_type=jax.ShapeDtypeStruct((n_rows, D), jnp.float32),
        compiler_params=pltpu.CompilerParams(
            needs_layout_passes=False, use_tc_tiling_on_sc=False),
        scratch_types=[
            pltpu.VMEM((per_w,), jnp.int32),
            pltpu.VMEM((per_w,), jnp.int32),
            pltpu.VMEM((n_small, D), jnp.float32),
            pltpu.VMEM((CHUNK, D), jnp.float32),
            pltpu.VMEM((D, L), jnp.float32),
            pltpu.VMEM((D,), jnp.float32),
            pltpu.VMEM((D,), jnp.float32),
            pltpu.SemaphoreType.DMA,
        ],
    )
    def k(aidx_hbm, iidx_hbm, atab_hbm, itab_hbm, g_hbm, bt_hbm, out_hbm,
          aidx_v, iidx_v, itab_v, arows_v, xbuf_v, g_v, bt_v, sem):
        wid = lax.axis_index("s") * 2 + lax.axis_index("c")
        base = wid * per_w

        pltpu.sync_copy(aidx_hbm.at[pl.ds(base, per_w)], aidx_v)
        pltpu.sync_copy(iidx_hbm.at[pl.ds(base, per_w)], iidx_v)
        pltpu.sync_copy(itab_hbm, itab_v)
        pltpu.sync_copy(g_hbm, g_v)
        pltpu.sync_copy(bt_hbm, bt_v)

        lanes = lax.iota(jnp.int32, L)
        gs = [g_v[pl.ds(c * L, L)] for c in range(D // L)]
        bs = [bt_v[pl.ds(c * L, L)] for c in range(D // L)]

        def chunk_body(ci, carry):
            loc = ci * CHUNK
            pltpu.async_copy(
                atab_hbm.at[aidx_v.at[pl.ds(loc, CHUNK)]], arows_v, sem
            ).wait()

            def group_body(g, gcarry):
                rows = g * L + lanes
                ivals = iidx_v[pl.ds(loc + g * L, L)]
                s = jnp.zeros((L,), jnp.float32)
                q = jnp.zeros((L,), jnp.float32)
                for j in range(D):
                    col = jnp.full((L,), j, jnp.int32)
                    a = plsc.load_gather(arows_v, [rows, col])
                    b = plsc.load_gather(itab_v, [ivals, col])
                    x = jnp.maximum(a + b, 0.0)
                    xbuf_v[j] = x
                    s = s + x
                    q = q + x * x
                mean = s * (1.0 / D)
                var = q * (1.0 / D) - mean * mean
                h = jnp.maximum(var, 0.0) + EPS
                iv = 0x5F3759DF - lax.shift_right_logical(
                    plsc.bitcast(h, jnp.int32), 1)
                y = plsc.bitcast(iv, jnp.float32)
                y = y * (1.5 - 0.5 * h * y * y)
                y = y * (1.5 - 0.5 * h * y * y)
                y = y * (1.5 - 0.5 * h * y * y)
                for j in range(D):
                    t = (xbuf_v[j] - mean) * y
                    o = t * gs[j // L][j % L] + bs[j // L][j % L]
                    plsc.store_scatter(
                        arows_v, [rows, jnp.full((L,), j, jnp.int32)], o)
                return gcarry

            lax.fori_loop(0, CHUNK // L, group_body, 0)
            pltpu.sync_copy(arows_v, out_hbm.at[pl.ds(base + loc, CHUNK)])
            return carry

        lax.fori_loop(0, n_chunks, chunk_body, 0)

    return k


def kernel(apiid, interval, apiid_table, interval_table, ln_gamma, ln_beta):
    b, s = apiid.shape
    n = b * s
    call = _sc_embed_ln(n, interval_table.shape[0], 32)
    out = call(apiid.reshape(n), interval.reshape(n),
               apiid_table, interval_table, ln_gamma, ln_beta)
    return out.reshape(b, s, D)


# speedup vs baseline: 1.3145x; 1.2942x over previous
import functools

import jax
import jax.numpy as jnp
from jax import lax
from jax.experimental import pallas as pl
from jax.experimental.pallas import tpu as pltpu
from jax.experimental.pallas import tpu_sc as plsc

EPS = 1e-12
D = 64
L = 16
CHUNK = 128
NSUB = 4
SUB = CHUNK // NSUB


def _sc_embed_ln(n_rows, n_small, num_workers):
    per_w = n_rows // num_workers
    n_chunks = per_w // CHUNK
    mesh = plsc.VectorSubcoreMesh(core_axis_name="c", subcore_axis_name="s")

    @functools.partial(
        pl.kernel,
        mesh=mesh,
        out_type=jax.ShapeDtypeStruct((n_rows, D), jnp.float32),
        compiler_params=pltpu.CompilerParams(
            needs_layout_passes=False, use_tc_tiling_on_sc=False),
        scratch_types=[
            pltpu.VMEM((per_w,), jnp.int32),
            pltpu.VMEM((per_w,), jnp.int32),
            pltpu.VMEM((n_small, D), jnp.float32),
            pltpu.VMEM((CHUNK, D), jnp.float32),
            pltpu.VMEM((CHUNK, D), jnp.float32),
            pltpu.VMEM((D, L), jnp.float32),
            pltpu.VMEM((D,), jnp.float32),
            pltpu.VMEM((D,), jnp.float32),
            pltpu.SemaphoreType.DMA,
            pltpu.SemaphoreType.DMA,
            pltpu.SemaphoreType.DMA,
            pltpu.SemaphoreType.DMA,
        ],
    )
    def k(aidx_hbm, iidx_hbm, atab_hbm, itab_hbm, g_hbm, bt_hbm, out_hbm,
          aidx_v, iidx_v, itab_v, rows_a, rows_b, xbuf_v, g_v, bt_v,
          sem_ga, sem_gb, sem_wa, sem_wb):
        wid = lax.axis_index("s") * 2 + lax.axis_index("c")
        base = wid * per_w

        pltpu.sync_copy(aidx_hbm.at[pl.ds(base, per_w)], aidx_v)
        pltpu.sync_copy(iidx_hbm.at[pl.ds(base, per_w)], iidx_v)
        pltpu.sync_copy(itab_hbm, itab_v)
        pltpu.sync_copy(g_hbm, g_v)
        pltpu.sync_copy(bt_hbm, bt_v)

        lanes = lax.iota(jnp.int32, L)
        gs = [g_v[pl.ds(c * L, L)] for c in range(D // L)]
        bs = [bt_v[pl.ds(c * L, L)] for c in range(D // L)]

        def gstart(ci, buf, sem):
            for si in range(NSUB):
                pltpu.async_copy(
                    atab_hbm.at[aidx_v.at[pl.ds(ci * CHUNK + si * SUB, SUB)]],
                    buf.at[pl.ds(si * SUB, SUB)], sem)

        def gwait(buf, sem):
            for si in range(NSUB):
                pltpu.make_async_copy(
                    atab_hbm.at[aidx_v.at[pl.ds(0, SUB)]],
                    buf.at[pl.ds(si * SUB, SUB)], sem).wait()

        def wstart(ci, buf, sem):
            pltpu.async_copy(buf, out_hbm.at[pl.ds(base + ci * CHUNK, CHUNK)],
                             sem)

        def wwait(buf, sem):
            pltpu.make_async_copy(
                buf, out_hbm.at[pl.ds(base, CHUNK)], sem).wait()

        ones = jnp.ones((L,), jnp.float32)
        zeros = jnp.zeros((L,), jnp.float32)
        plain = jnp.bool_(True)
        for c in range(D // L):
            plain = plain & jnp.all(gs[c] == ones) & jnp.all(bs[c] == zeros)

        PB = 4

        def compute(ci, arows_v):
            loc = ci * CHUNK

            def group_body(g, gcarry):
                rows = g * L + lanes
                ivals = iidx_v[pl.ds(loc + g * L, L)]
                ss = [jnp.zeros((L,), jnp.float32) for _ in range(PB)]
                qq = [jnp.zeros((L,), jnp.float32) for _ in range(PB)]
                for j0 in range(0, D, PB):
                    avs = [plsc.load_gather(
                        arows_v, [rows, jnp.full((L,), j0 + t, jnp.int32)])
                        for t in range(PB)]
                    bvs = [plsc.load_gather(
                        itab_v, [ivals, jnp.full((L,), j0 + t, jnp.int32)])
                        for t in range(PB)]
                    for t in range(PB):
                        x = jnp.maximum(avs[t] + bvs[t], 0.0)
                        xbuf_v[j0 + t] = x
                        ss[t] = ss[t] + x
                        qq[t] = qq[t] + x * x
                s = (ss[0] + ss[1]) + (ss[2] + ss[3])
                q = (qq[0] + qq[1]) + (qq[2] + qq[3])
                mean = s * (1.0 / D)
                var = q * (1.0 / D) - mean * mean
                h = jnp.maximum(var, 0.0) + EPS
                iv = 0x5F3759DF - lax.shift_right_logical(
                    plsc.bitcast(h, jnp.int32), 1)
                y = plsc.bitcast(iv, jnp.float32)
                y = y * (1.5 - 0.5 * h * y * y)
                y = y * (1.5 - 0.5 * h * y * y)
                y = y * (1.5 - 0.5 * h * y * y)

                @pl.when(plain)
                def _():
                    for j0 in range(0, D, PB):
                        xs_ = [xbuf_v[j0 + t] for t in range(PB)]
                        for t in range(PB):
                            o = (xs_[t] - mean) * y
                            plsc.store_scatter(
                                arows_v,
                                [rows, jnp.full((L,), j0 + t, jnp.int32)], o)

                @pl.when(jnp.logical_not(plain))
                def _():
                    for j0 in range(0, D, PB):
                        xs_ = [xbuf_v[j0 + t] for t in range(PB)]
                        for t in range(PB):
                            j = j0 + t
                            o = ((xs_[t] - mean) * y * gs[j // L][j % L]
                                 + bs[j // L][j % L])
                            plsc.store_scatter(
                                arows_v,
                                [rows, jnp.full((L,), j, jnp.int32)], o)

                return gcarry

            lax.fori_loop(0, CHUNK // L, group_body, 0)

        n_pairs = n_chunks // 2
        gstart(0, rows_a, sem_ga)

        def pair_body(pi, carry):
            i2 = pi * 2
            gwait(rows_a, sem_ga)

            @pl.when(pi > 0)
            def _():
                wwait(rows_b, sem_wb)

            gstart(i2 + 1, rows_b, sem_gb)
            compute(i2, rows_a)
            wstart(i2, rows_a, sem_wa)
            gwait(rows_b, sem_gb)

            @pl.when(pi < n_pairs - 1)
            def _():
                wwait(rows_a, sem_wa)
                gstart(i2 + 2, rows_a, sem_ga)

            compute(i2 + 1, rows_b)
            wstart(i2 + 1, rows_b, sem_wb)
            return carry

        lax.fori_loop(0, n_pairs, pair_body, 0)
        wwait(rows_a, sem_wa)
        wwait(rows_b, sem_wb)

    return k


def kernel(apiid, interval, apiid_table, interval_table, ln_gamma, ln_beta):
    b, s = apiid.shape
    n = b * s
    call = _sc_embed_ln(n, interval_table.shape[0], 32)
    out = call(apiid.reshape(n), interval.reshape(n),
               apiid_table, interval_table, ln_gamma, ln_beta)
    return out.reshape(b, s, D)


# speedup vs baseline: 1.3268x; 1.0093x over previous
import functools

import jax
import jax.numpy as jnp
from jax import lax
from jax.experimental import pallas as pl
from jax.experimental.pallas import tpu as pltpu
from jax.experimental.pallas import tpu_sc as plsc

EPS = 1e-12
D = 64
DP = 128
L = 16
CHUNK = 128
NSUB = 4
SUB = CHUNK // NSUB


def _sc_embed_ln(n_rows, n_small, num_workers):
    per_w = n_rows // num_workers
    n_chunks = per_w // CHUNK
    mesh = plsc.VectorSubcoreMesh(core_axis_name="c", subcore_axis_name="s")

    @functools.partial(
        pl.kernel,
        mesh=mesh,
        out_type=jax.ShapeDtypeStruct((n_rows, DP), jnp.float32),
        compiler_params=pltpu.CompilerParams(
            needs_layout_passes=False, use_tc_tiling_on_sc=False),
        scratch_types=[
            pltpu.VMEM((per_w,), jnp.int32),
            pltpu.VMEM((per_w,), jnp.int32),
            pltpu.VMEM((n_small, D), jnp.float32),
            pltpu.VMEM((CHUNK, D), jnp.float32),
            pltpu.VMEM((CHUNK, D), jnp.float32),
            pltpu.VMEM((CHUNK, DP), jnp.float32),
            pltpu.VMEM((CHUNK, DP), jnp.float32),
            pltpu.VMEM((D, L), jnp.float32),
            pltpu.VMEM((D,), jnp.float32),
            pltpu.VMEM((D,), jnp.float32),
            pltpu.SemaphoreType.DMA,
            pltpu.SemaphoreType.DMA,
            pltpu.SemaphoreType.DMA,
            pltpu.SemaphoreType.DMA,
        ],
    )
    def k(aidx_hbm, iidx_hbm, atab_hbm, itab_hbm, g_hbm, bt_hbm, out_hbm,
          aidx_v, iidx_v, itab_v, rows_a, rows_b, outs_a, outs_b,
          xbuf_v, g_v, bt_v, sem_ga, sem_gb, sem_wa, sem_wb):
        wid = lax.axis_index("s") * 2 + lax.axis_index("c")
        base = wid * per_w

        pltpu.sync_copy(aidx_hbm.at[pl.ds(base, per_w)], aidx_v)
        pltpu.sync_copy(iidx_hbm.at[pl.ds(base, per_w)], iidx_v)
        pltpu.sync_copy(itab_hbm, itab_v)
        pltpu.sync_copy(g_hbm, g_v)
        pltpu.sync_copy(bt_hbm, bt_v)

        lanes = lax.iota(jnp.int32, L)
        gs = [g_v[pl.ds(c * L, L)] for c in range(D // L)]
        bs = [bt_v[pl.ds(c * L, L)] for c in range(D // L)]

        def gstart(ci, buf, sem):
            for si in range(NSUB):
                pltpu.async_copy(
                    atab_hbm.at[aidx_v.at[pl.ds(ci * CHUNK + si * SUB, SUB)]],
                    buf.at[pl.ds(si * SUB, SUB)], sem)

        def gwait(buf, sem):
            for si in range(NSUB):
                pltpu.make_async_copy(
                    atab_hbm.at[aidx_v.at[pl.ds(0, SUB)]],
                    buf.at[pl.ds(si * SUB, SUB)], sem).wait()

        def wstart(ci, buf, sem):
            pltpu.async_copy(buf, out_hbm.at[pl.ds(base + ci * CHUNK, CHUNK)],
                             sem)

        def wwait(buf, sem):
            pltpu.make_async_copy(
                buf, out_hbm.at[pl.ds(base, CHUNK)], sem).wait()

        ones = jnp.ones((L,), jnp.float32)
        zeros = jnp.zeros((L,), jnp.float32)
        plain = jnp.bool_(True)
        for c in range(D // L):
            plain = plain & jnp.all(gs[c] == ones) & jnp.all(bs[c] == zeros)

        PB = 4

        def compute(ci, arows_v, outs_v):
            loc = ci * CHUNK

            def group_body(g, gcarry):
                rows = g * L + lanes
                ivals = iidx_v[pl.ds(loc + g * L, L)]
                ss = [jnp.zeros((L,), jnp.float32) for _ in range(PB)]
                qq = [jnp.zeros((L,), jnp.float32) for _ in range(PB)]
                for j0 in range(0, D, PB):
                    avs = [plsc.load_gather(
                        arows_v, [rows, jnp.full((L,), j0 + t, jnp.int32)])
                        for t in range(PB)]
                    bvs = [plsc.load_gather(
                        itab_v, [ivals, jnp.full((L,), j0 + t, jnp.int32)])
                        for t in range(PB)]
                    for t in range(PB):
                        x = jnp.maximum(avs[t] + bvs[t], 0.0)
                        xbuf_v[j0 + t] = x
                        ss[t] = ss[t] + x
                        qq[t] = qq[t] + x * x
                s = (ss[0] + ss[1]) + (ss[2] + ss[3])
                q = (qq[0] + qq[1]) + (qq[2] + qq[3])
                mean = s * (1.0 / D)
                var = q * (1.0 / D) - mean * mean
                h = jnp.maximum(var, 0.0) + EPS
                iv = 0x5F3759DF - lax.shift_right_logical(
                    plsc.bitcast(h, jnp.int32), 1)
                y = plsc.bitcast(iv, jnp.float32)
                y = y * (1.5 - 0.5 * h * y * y)
                y = y * (1.5 - 0.5 * h * y * y)
                y = y * (1.5 - 0.5 * h * y * y)

                @pl.when(plain)
                def _():
                    for j0 in range(0, D, PB):
                        xs_ = [xbuf_v[j0 + t] for t in range(PB)]
                        for t in range(PB):
                            o = (xs_[t] - mean) * y
                            plsc.store_scatter(
                                outs_v,
                                [rows, jnp.full((L,), j0 + t, jnp.int32)], o)

                @pl.when(jnp.logical_not(plain))
                def _():
                    for j0 in range(0, D, PB):
                        xs_ = [xbuf_v[j0 + t] for t in range(PB)]
                        for t in range(PB):
                            j = j0 + t
                            o = ((xs_[t] - mean) * y * gs[j // L][j % L]
                                 + bs[j // L][j % L])
                            plsc.store_scatter(
                                outs_v,
                                [rows, jnp.full((L,), j, jnp.int32)], o)

                return gcarry

            lax.fori_loop(0, CHUNK // L, group_body, 0)

        n_pairs = n_chunks // 2
        gstart(0, rows_a, sem_ga)

        def pair_body(pi, carry):
            i2 = pi * 2
            gwait(rows_a, sem_ga)
            gstart(i2 + 1, rows_b, sem_gb)

            @pl.when(pi > 0)
            def _():
                wwait(outs_a, sem_wa)

            compute(i2, rows_a, outs_a)
            wstart(i2, outs_a, sem_wa)
            gwait(rows_b, sem_gb)

            @pl.when(pi < n_pairs - 1)
            def _():
                gstart(i2 + 2, rows_a, sem_ga)

            @pl.when(pi > 0)
            def _():
                wwait(outs_b, sem_wb)

            compute(i2 + 1, rows_b, outs_b)
            wstart(i2 + 1, outs_b, sem_wb)
            return carry

        lax.fori_loop(0, n_pairs, pair_body, 0)
        wwait(outs_a, sem_wa)
        wwait(outs_b, sem_wb)

    return k


def _tc_retile(b, s):
    bm = 128
    flat = s * DP

    def body(in_ref, out_ref):
        x = in_ref[...].reshape(bm * s, DP)
        out_ref[...] = x[:, :D].reshape(bm, s, D)

    return pl.pallas_call(
        body,
        grid=(b // bm,),
        in_specs=[pl.BlockSpec((bm * flat,), lambda i: (i,))],
        out_specs=pl.BlockSpec((bm, s, D), lambda i: (i, 0, 0)),
        out_shape=jax.ShapeDtypeStruct((b, s, D), jnp.float32),
    )


def kernel(apiid, interval, apiid_table, interval_table, ln_gamma, ln_beta):
    b, s = apiid.shape
    n = b * s
    call = _sc_embed_ln(n, interval_table.shape[0], 32)
    padded = call(apiid.reshape(n), interval.reshape(n),
                  apiid_table, interval_table, ln_gamma, ln_beta)
    return _tc_retile(b, s)(padded.reshape(n * DP))
